# baseline (speedup 1.0000x reference)
import jax
import jax.numpy as jnp
from jax.experimental import pallas as pl

T, K, D, F, E = 32, 8, 1024, 512, 64


EB = 4


def _moe_kernel(ids_ref, w_ref, x_ref, wg_ref, wu_ref, wd_ref, o_ref):
    step = pl.program_id(0)

    @pl.when(step == 0)
    def _init():
        o_ref[...] = jnp.zeros_like(o_ref)

    x = x_ref[...]
    acc = jnp.zeros((T, D), jnp.float32)
    for j in range(EB):
        e = step * EB + j
        mask = (ids_ref[...] == e).astype(jnp.float32)
        c = jnp.sum(w_ref[...] * mask, axis=1)
        g = jax.lax.dot_general(x, wg_ref[j], (((1,), (1,)), ((), ())),
                                preferred_element_type=jnp.float32)
        u = jax.lax.dot_general(x, wu_ref[j], (((1,), (1,)), ((), ())),
                                preferred_element_type=jnp.float32)
        act = (g * jax.nn.sigmoid(g)) * u
        eo = jax.lax.dot_general(act, wd_ref[j], (((1,), (1,)), ((), ())),
                                 preferred_element_type=jnp.float32)
        acc = acc + eo * c[:, None]
    o_ref[...] += acc


def kernel(hidden_states, top_k_index, top_k_weights, w_gate, w_up, w_down):
    out = pl.pallas_call(
        _moe_kernel,
        grid=(E // EB,),
        in_specs=[
            pl.BlockSpec((T, K), lambda e: (0, 0)),
            pl.BlockSpec((T, K), lambda e: (0, 0)),
            pl.BlockSpec((T, D), lambda e: (0, 0)),
            pl.BlockSpec((EB, F, D), lambda e: (e, 0, 0)),
            pl.BlockSpec((EB, F, D), lambda e: (e, 0, 0)),
            pl.BlockSpec((EB, D, F), lambda e: (e, 0, 0)),
        ],
        out_specs=pl.BlockSpec((T, D), lambda e: (0, 0)),
        out_shape=jax.ShapeDtypeStruct((T, D), jnp.float32),
    )(top_k_index, top_k_weights, hidden_states, w_gate, w_up, w_down)
    return out


# speedup vs baseline: 1.0084x; 1.0084x over previous
import jax
import jax.numpy as jnp
from jax.experimental import pallas as pl

T, K, D, F, E = 32, 8, 1024, 512, 64


EB = 2


def _moe_kernel(ids_ref, w_ref, x_ref, wg_ref, wu_ref, wd_ref, o_ref):
    step = pl.program_id(0)

    @pl.when(step == 0)
    def _init():
        o_ref[...] = jnp.zeros_like(o_ref)

    x = x_ref[...]
    acc = jnp.zeros((T, D), jnp.float32)
    for j in range(EB):
        e = step * EB + j
        mask = (ids_ref[...] == e).astype(jnp.float32)
        c = jnp.sum(w_ref[...] * mask, axis=1)
        g = jax.lax.dot_general(x, wg_ref[j], (((1,), (1,)), ((), ())),
                                preferred_element_type=jnp.float32)
        u = jax.lax.dot_general(x, wu_ref[j], (((1,), (1,)), ((), ())),
                                preferred_element_type=jnp.float32)
        act = (g * jax.nn.sigmoid(g)) * u
        eo = jax.lax.dot_general(act, wd_ref[j], (((1,), (1,)), ((), ())),
                                 preferred_element_type=jnp.float32)
        acc = acc + eo * c[:, None]
    o_ref[...] += acc


def kernel(hidden_states, top_k_index, top_k_weights, w_gate, w_up, w_down):
    out = pl.pallas_call(
        _moe_kernel,
        grid=(E // EB,),
        in_specs=[
            pl.BlockSpec((T, K), lambda e: (0, 0)),
            pl.BlockSpec((T, K), lambda e: (0, 0)),
            pl.BlockSpec((T, D), lambda e: (0, 0)),
            pl.BlockSpec((EB, F, D), lambda e: (e, 0, 0)),
            pl.BlockSpec((EB, F, D), lambda e: (e, 0, 0)),
            pl.BlockSpec((EB, D, F), lambda e: (e, 0, 0)),
        ],
        out_specs=pl.BlockSpec((T, D), lambda e: (0, 0)),
        out_shape=jax.ShapeDtypeStruct((T, D), jnp.float32),
    )(top_k_index, top_k_weights, hidden_states, w_gate, w_up, w_down)
    return out
